# baseline (speedup 1.0000x reference)
import functools

import jax
import jax.numpy as jnp
from jax import lax
from jax.experimental import pallas as pl
from jax.experimental.pallas import tpu as pltpu
from jax.experimental.pallas import tpu_sc as plsc

_N = 10000
_E = 320000
_D = 128
_NC = 2
_NS = 16
_NW = _NC * _NS
_NB = 4
_CH = 40
_SLOWC = 0
_KS = 168
_KF = 336
_EPAD = _NS * (_KS + _KF) * _CH
_CHD = 128
_KD = 80
_NBD = 4
_EPADD = _NW * _KD * _CHD
_NP = 10112
_RPT = _NP // _NS


def _sc_mesh():
    return plsc.VectorSubcoreMesh(
        core_axis_name="c", subcore_axis_name="s",
        num_cores=_NC, num_subcores=_NS)


def _sc_aggregate(ml, src_t, dst_t, zeros_nd):
    @functools.partial(
        pl.kernel,
        out_type=jax.ShapeDtypeStruct((_NC, _NP, _D), jnp.float32),
        mesh=_sc_mesh(),
        scratch_types=[
            pltpu.VMEM((_KF * _CH,), jnp.int32),
            pltpu.VMEM((_KF * _CH,), jnp.int32),
            [pltpu.VMEM((_CH, _D), jnp.float32) for _ in range(_NB)],
            pltpu.VMEM_SHARED((_NP, _D), jnp.float32),
            [pltpu.SemaphoreType.DMA for _ in range(_NB)],
            [pltpu.SemaphoreType.DMA for _ in range(_NB)],
        ],
    )
    def run(ml_hbm, src_hbm, dst_hbm, zz_hbm, out_hbm,
            src_v, dst_v, rows, agg_sh, gsems, ssems):
        c = lax.axis_index("c")
        s = lax.axis_index("s")
        wid = s * _NC + c
        kc = lax.select(c == _SLOWC, _KS, _KF)
        pltpu.sync_copy(src_hbm.at[wid], src_v)
        pltpu.sync_copy(dst_hbm.at[wid], dst_v)
        pltpu.sync_copy(zz_hbm.at[pl.ds(s * _RPT, _RPT)],
                        agg_sh.at[pl.ds(s * _RPT, _RPT)])
        for b in range(_NB):
            pltpu.async_copy(
                ml_hbm.at[src_v.at[pl.ds(b * _CH, _CH)]], rows[b], gsems[b])
        plsc.subcore_barrier()

        def step(t, carry):
            jj = t * _NB
            for b in range(_NB):
                j = jj + b
                pltpu.make_async_copy(
                    ml_hbm.at[src_v.at[pl.ds(j * _CH, _CH)]],
                    rows[b], gsems[b]).wait()
                pltpu.async_copy(
                    rows[b], agg_sh.at[dst_v.at[pl.ds(j * _CH, _CH)]],
                    ssems[b], add=True)
            for b in range(_NB):
                j = jj + b
                pltpu.make_async_copy(
                    rows[b], agg_sh.at[dst_v.at[pl.ds(j * _CH, _CH)]],
                    ssems[b]).wait()
                nxt = j + _NB

                @pl.when(nxt < kc)
                def _():
                    pltpu.async_copy(
                        ml_hbm.at[src_v.at[pl.ds(nxt * _CH, _CH)]],
                        rows[b], gsems[b])
            return carry

        lax.fori_loop(0, kc // _NB, step, 0)
        plsc.subcore_barrier()
        pltpu.sync_copy(agg_sh.at[pl.ds(s * _RPT, _RPT)],
                        out_hbm.at[c, pl.ds(s * _RPT, _RPT)])

    return run(ml, src_t, dst_t, zeros_nd)


def _sc_degree(dst_t, zeros_nd, ones_t):
    @functools.partial(
        pl.kernel,
        out_type=jax.ShapeDtypeStruct((_NC, _NP, _D), jnp.float32),
        mesh=_sc_mesh(),
        scratch_types=[
            pltpu.VMEM((_KD, _CHD), jnp.int32),
            pltpu.VMEM((_CHD, _D), jnp.float32),
            pltpu.VMEM_SHARED((_NP, _D), jnp.float32),
            [pltpu.SemaphoreType.DMA for _ in range(_NBD)],
        ],
    )
    def run(dst_hbm, zz_hbm, ones_hbm, out_hbm, dst_v, ones_v, deg_sh, ssems):
        c = lax.axis_index("c")
        s = lax.axis_index("s")
        wid = s * _NC + c
        pltpu.sync_copy(dst_hbm.at[wid], dst_v)
        pltpu.sync_copy(ones_hbm, ones_v)
        pltpu.sync_copy(zz_hbm.at[pl.ds(s * _RPT, _RPT)],
                        deg_sh.at[pl.ds(s * _RPT, _RPT)])
        plsc.subcore_barrier()
        for b in range(_NBD):
            pltpu.async_copy(ones_v, deg_sh.at[dst_v.at[b]], ssems[b],
                             add=True)

        def step(t, carry):
            jj = t * _NBD
            for b in range(_NBD):
                j = jj + b
                pltpu.make_async_copy(
                    ones_v, deg_sh.at[dst_v.at[j]], ssems[b]).wait()
                nxt = j + _NBD

                @pl.when(nxt < _KD)
                def _():
                    pltpu.async_copy(
                        ones_v, deg_sh.at[dst_v.at[nxt]], ssems[b], add=True)
            return carry

        lax.fori_loop(0, _KD // _NBD, step, 0)
        plsc.subcore_barrier()
        pltpu.sync_copy(deg_sh.at[pl.ds(s * _RPT, _RPT)],
                        out_hbm.at[c, pl.ds(s * _RPT, _RPT)])

    return run(dst_t, zeros_nd, ones_t)


def _tc_linear(h, Wl, Wr, b2):
    def body(h_ref, wl_ref, wr_ref, b_ref, ml_ref, hr_ref):
        hh = h_ref[...]
        ml_ref[...] = jnp.dot(hh, wl_ref[...],
                              preferred_element_type=jnp.float32)
        hr_ref[...] = jnp.dot(hh, wr_ref[...],
                              preferred_element_type=jnp.float32) + b_ref[...]

    return pl.pallas_call(
        body,
        out_shape=(jax.ShapeDtypeStruct((_N, _D), jnp.float32),
                   jax.ShapeDtypeStruct((_N, _D), jnp.float32)),
    )(h, Wl, Wr, b2)


def _tc_combine(a0, a1, d0, d1, hr):
    def body(a0_ref, a1_ref, d0_ref, d1_ref, hr_ref, o_ref):
        agg = a0_ref[0:_N, :] + a1_ref[0:_N, :]
        deg = d0_ref[0:_N, 0:1] + d1_ref[0:_N, 0:1]
        rdeg = 1.0 / jnp.maximum(deg, 1.0)
        o_ref[...] = jnp.maximum(agg * rdeg + hr_ref[...], 0.0)

    return pl.pallas_call(
        body,
        out_shape=jax.ShapeDtypeStruct((_N, _D), jnp.float32),
    )(a0, a1, d0, d1, hr)


def _tc_combine_linear(a0, a1, d0, d1, hr, Wl, Wr, b2):
    def body(a0_ref, a1_ref, d0_ref, d1_ref, hr_ref, wl_ref, wr_ref, b_ref,
             ml_ref, hr2_ref):
        agg = a0_ref[0:_N, :] + a1_ref[0:_N, :]
        deg = d0_ref[0:_N, 0:1] + d1_ref[0:_N, 0:1]
        rdeg = 1.0 / jnp.maximum(deg, 1.0)
        h = jnp.maximum(agg * rdeg + hr_ref[...], 0.0)
        ml_ref[...] = jnp.dot(h, wl_ref[...],
                              preferred_element_type=jnp.float32)
        hr2_ref[...] = jnp.dot(h, wr_ref[...],
                               preferred_element_type=jnp.float32) + b_ref[...]

    return pl.pallas_call(
        body,
        out_shape=(jax.ShapeDtypeStruct((_N, _D), jnp.float32),
                   jax.ShapeDtypeStruct((_N, _D), jnp.float32)),
    )(a0, a1, d0, d1, hr, Wl, Wr, b2)


def kernel(x, edge_index, Wl0, Wr0, b0, Wl1, Wr1, b1,
           Wl2, Wr2, b2, Wl3, Wr3, b3):
    src = edge_index[0]
    dst = edge_index[1]

    def _layout(idx, fill):
        pad = _EPAD - _E
        flat = jnp.concatenate([idx, jnp.full((pad,), fill, jnp.int32)])
        ls = _KS * _CH
        lf = _KF * _CH
        slow = flat[:_NS * ls].reshape(_NS, ls)
        slow = jnp.concatenate(
            [slow, jnp.full((_NS, lf - ls), fill, jnp.int32)], axis=1)
        fast = flat[_NS * ls:].reshape(_NS, lf)
        parts = (slow, fast) if _SLOWC == 0 else (fast, slow)
        return jnp.stack(parts, axis=1).reshape(_NW, lf)

    src_t = _layout(src, 0)
    dst_t = _layout(dst, _N)
    padd = _EPADD - _E
    dst_td = jnp.concatenate(
        [dst, jnp.full((padd,), _N, jnp.int32)]).reshape(_NW, _KD, _CHD)
    zeros_nd = jnp.zeros((_NP, _D), jnp.float32)
    ones_t = jnp.ones((_CHD, _D), jnp.float32)

    dg = _sc_degree(dst_td, zeros_nd, ones_t)

    ml, hr = _tc_linear(x, Wl0, Wr0, b0.reshape(1, _D))
    nxt = ((Wl1, Wr1, b1), (Wl2, Wr2, b2), (Wl3, Wr3, b3))
    for i in range(4):
        a = _sc_aggregate(ml, src_t, dst_t, zeros_nd)
        if i < 3:
            Wl, Wr, b = nxt[i]
            ml, hr = _tc_combine_linear(a[0], a[1], dg[0], dg[1], hr,
                                        Wl, Wr, b.reshape(1, _D))
        else:
            h = _tc_combine(a[0], a[1], dg[0], dg[1], hr)
    return h


# speedup vs baseline: 1.0454x; 1.0454x over previous
import functools

import jax
import jax.numpy as jnp
from jax import lax
from jax.experimental import pallas as pl
from jax.experimental.pallas import tpu as pltpu
from jax.experimental.pallas import tpu_sc as plsc

_N = 10000
_E = 320000
_D = 128
_NC = 2
_NS = 16
_NW = _NC * _NS
_NB = 4
_CH = 40
_SLOWC = 1
_KS = 168
_KF = 336
_EPAD = _NS * (_KS + _KF) * _CH
_CHD = 128
_KD = 80
_NBD = 4
_EPADD = _NW * _KD * _CHD
_NP = 10112
_RPT = _NP // _NS


def _sc_mesh():
    return plsc.VectorSubcoreMesh(
        core_axis_name="c", subcore_axis_name="s",
        num_cores=_NC, num_subcores=_NS)


def _sc_aggregate(ml, src_t, dst_t, zeros_nd):
    @functools.partial(
        pl.kernel,
        out_type=jax.ShapeDtypeStruct((_NC, _NP, _D), jnp.float32),
        mesh=_sc_mesh(),
        scratch_types=[
            pltpu.VMEM((_KF * _CH,), jnp.int32),
            pltpu.VMEM((_KF * _CH,), jnp.int32),
            [pltpu.VMEM((_CH, _D), jnp.float32) for _ in range(_NB)],
            pltpu.VMEM_SHARED((_NP, _D), jnp.float32),
            [pltpu.SemaphoreType.DMA for _ in range(_NB)],
            [pltpu.SemaphoreType.DMA for _ in range(_NB)],
        ],
    )
    def run(ml_hbm, src_hbm, dst_hbm, zz_hbm, out_hbm,
            src_v, dst_v, rows, agg_sh, gsems, ssems):
        c = lax.axis_index("c")
        s = lax.axis_index("s")
        wid = s * _NC + c
        kc = lax.select(c == _SLOWC, _KS, _KF)
        pltpu.sync_copy(src_hbm.at[wid], src_v)
        pltpu.sync_copy(dst_hbm.at[wid], dst_v)
        pltpu.sync_copy(zz_hbm.at[pl.ds(s * _RPT, _RPT)],
                        agg_sh.at[pl.ds(s * _RPT, _RPT)])
        for b in range(_NB):
            pltpu.async_copy(
                ml_hbm.at[src_v.at[pl.ds(b * _CH, _CH)]], rows[b], gsems[b])
        plsc.subcore_barrier()

        def step(t, carry):
            jj = t * _NB
            for b in range(_NB):
                j = jj + b
                pltpu.make_async_copy(
                    ml_hbm.at[src_v.at[pl.ds(j * _CH, _CH)]],
                    rows[b], gsems[b]).wait()
                pltpu.async_copy(
                    rows[b], agg_sh.at[dst_v.at[pl.ds(j * _CH, _CH)]],
                    ssems[b], add=True)
            for b in range(_NB):
                j = jj + b
                pltpu.make_async_copy(
                    rows[b], agg_sh.at[dst_v.at[pl.ds(j * _CH, _CH)]],
                    ssems[b]).wait()
                nxt = j + _NB

                @pl.when(nxt < kc)
                def _():
                    pltpu.async_copy(
                        ml_hbm.at[src_v.at[pl.ds(nxt * _CH, _CH)]],
                        rows[b], gsems[b])
            return carry

        lax.fori_loop(0, kc // _NB, step, 0)
        plsc.subcore_barrier()
        pltpu.sync_copy(agg_sh.at[pl.ds(s * _RPT, _RPT)],
                        out_hbm.at[c, pl.ds(s * _RPT, _RPT)])

    return run(ml, src_t, dst_t, zeros_nd)


def _sc_degree(dst_t, zeros_nd, ones_t):
    @functools.partial(
        pl.kernel,
        out_type=jax.ShapeDtypeStruct((_NC, _NP, _D), jnp.float32),
        mesh=_sc_mesh(),
        scratch_types=[
            pltpu.VMEM((_KD, _CHD), jnp.int32),
            pltpu.VMEM((_CHD, _D), jnp.float32),
            pltpu.VMEM_SHARED((_NP, _D), jnp.float32),
            [pltpu.SemaphoreType.DMA for _ in range(_NBD)],
        ],
    )
    def run(dst_hbm, zz_hbm, ones_hbm, out_hbm, dst_v, ones_v, deg_sh, ssems):
        c = lax.axis_index("c")
        s = lax.axis_index("s")
        wid = s * _NC + c
        pltpu.sync_copy(dst_hbm.at[wid], dst_v)
        pltpu.sync_copy(ones_hbm, ones_v)
        pltpu.sync_copy(zz_hbm.at[pl.ds(s * _RPT, _RPT)],
                        deg_sh.at[pl.ds(s * _RPT, _RPT)])
        plsc.subcore_barrier()
        for b in range(_NBD):
            pltpu.async_copy(ones_v, deg_sh.at[dst_v.at[b]], ssems[b],
                             add=True)

        def step(t, carry):
            jj = t * _NBD
            for b in range(_NBD):
                j = jj + b
                pltpu.make_async_copy(
                    ones_v, deg_sh.at[dst_v.at[j]], ssems[b]).wait()
                nxt = j + _NBD

                @pl.when(nxt < _KD)
                def _():
                    pltpu.async_copy(
                        ones_v, deg_sh.at[dst_v.at[nxt]], ssems[b], add=True)
            return carry

        lax.fori_loop(0, _KD // _NBD, step, 0)
        plsc.subcore_barrier()
        pltpu.sync_copy(deg_sh.at[pl.ds(s * _RPT, _RPT)],
                        out_hbm.at[c, pl.ds(s * _RPT, _RPT)])

    return run(dst_t, zeros_nd, ones_t)


def _tc_linear(h, Wl, Wr, b2):
    def body(h_ref, wl_ref, wr_ref, b_ref, ml_ref, hr_ref):
        hh = h_ref[...]
        ml_ref[...] = jnp.dot(hh, wl_ref[...],
                              preferred_element_type=jnp.float32)
        hr_ref[...] = jnp.dot(hh, wr_ref[...],
                              preferred_element_type=jnp.float32) + b_ref[...]

    return pl.pallas_call(
        body,
        out_shape=(jax.ShapeDtypeStruct((_N, _D), jnp.float32),
                   jax.ShapeDtypeStruct((_N, _D), jnp.float32)),
    )(h, Wl, Wr, b2)


def _tc_combine(a0, a1, d0, d1, hr):
    def body(a0_ref, a1_ref, d0_ref, d1_ref, hr_ref, o_ref):
        agg = a0_ref[0:_N, :] + a1_ref[0:_N, :]
        deg = d0_ref[0:_N, 0:1] + d1_ref[0:_N, 0:1]
        rdeg = 1.0 / jnp.maximum(deg, 1.0)
        o_ref[...] = jnp.maximum(agg * rdeg + hr_ref[...], 0.0)

    return pl.pallas_call(
        body,
        out_shape=jax.ShapeDtypeStruct((_N, _D), jnp.float32),
    )(a0, a1, d0, d1, hr)


def _tc_combine_linear(a0, a1, d0, d1, hr, Wl, Wr, b2):
    def body(a0_ref, a1_ref, d0_ref, d1_ref, hr_ref, wl_ref, wr_ref, b_ref,
             ml_ref, hr2_ref):
        agg = a0_ref[0:_N, :] + a1_ref[0:_N, :]
        deg = d0_ref[0:_N, 0:1] + d1_ref[0:_N, 0:1]
        rdeg = 1.0 / jnp.maximum(deg, 1.0)
        h = jnp.maximum(agg * rdeg + hr_ref[...], 0.0)
        ml_ref[...] = jnp.dot(h, wl_ref[...],
                              preferred_element_type=jnp.float32)
        hr2_ref[...] = jnp.dot(h, wr_ref[...],
                               preferred_element_type=jnp.float32) + b_ref[...]

    return pl.pallas_call(
        body,
        out_shape=(jax.ShapeDtypeStruct((_N, _D), jnp.float32),
                   jax.ShapeDtypeStruct((_N, _D), jnp.float32)),
    )(a0, a1, d0, d1, hr, Wl, Wr, b2)


def kernel(x, edge_index, Wl0, Wr0, b0, Wl1, Wr1, b1,
           Wl2, Wr2, b2, Wl3, Wr3, b3):
    src = edge_index[0]
    dst = edge_index[1]

    def _layout(idx, fill):
        pad = _EPAD - _E
        flat = jnp.concatenate([idx, jnp.full((pad,), fill, jnp.int32)])
        ls = _KS * _CH
        lf = _KF * _CH
        slow = flat[:_NS * ls].reshape(_NS, ls)
        slow = jnp.concatenate(
            [slow, jnp.full((_NS, lf - ls), fill, jnp.int32)], axis=1)
        fast = flat[_NS * ls:].reshape(_NS, lf)
        parts = (slow, fast) if _SLOWC == 0 else (fast, slow)
        return jnp.stack(parts, axis=1).reshape(_NW, lf)

    src_t = _layout(src, 0)
    dst_t = _layout(dst, _N)
    padd = _EPADD - _E
    dst_td = jnp.concatenate(
        [dst, jnp.full((padd,), _N, jnp.int32)]).reshape(_NW, _KD, _CHD)
    zeros_nd = jnp.zeros((_NP, _D), jnp.float32)
    ones_t = jnp.ones((_CHD, _D), jnp.float32)

    dg = _sc_degree(dst_td, zeros_nd, ones_t)

    ml, hr = _tc_linear(x, Wl0, Wr0, b0.reshape(1, _D))
    nxt = ((Wl1, Wr1, b1), (Wl2, Wr2, b2), (Wl3, Wr3, b3))
    for i in range(4):
        a = _sc_aggregate(ml, src_t, dst_t, zeros_nd)
        if i < 3:
            Wl, Wr, b = nxt[i]
            ml, hr = _tc_combine_linear(a[0], a[1], dg[0], dg[1], hr,
                                        Wl, Wr, b.reshape(1, _D))
        else:
            h = _tc_combine(a[0], a[1], dg[0], dg[1], hr)
    return h


# speedup vs baseline: 1.1312x; 1.0821x over previous
import functools

import jax
import jax.numpy as jnp
from jax import lax
from jax.experimental import pallas as pl
from jax.experimental.pallas import tpu as pltpu
from jax.experimental.pallas import tpu_sc as plsc

_N = 10000
_E = 320000
_D = 128
_NC = 2
_NS = 16
_NW = _NC * _NS
_NB = 4
_CH = 56
_SLOWC = 0
_KS = 180
_KF = 180
_EPAD = _NS * (_KS + _KF) * _CH
_CHD = 128
_KD = 80
_NBD = 4
_EPADD = _NW * _KD * _CHD
_NP = 10112
_RPT = _NP // _NS


def _sc_mesh():
    return plsc.VectorSubcoreMesh(
        core_axis_name="c", subcore_axis_name="s",
        num_cores=_NC, num_subcores=_NS)


def _sc_aggregate(ml, src_t, dst_t, zeros_nd):
    @functools.partial(
        pl.kernel,
        out_type=jax.ShapeDtypeStruct((_NC, _NP, _D), jnp.float32),
        mesh=_sc_mesh(),
        scratch_types=[
            pltpu.VMEM((_KF * _CH,), jnp.int32),
            pltpu.VMEM((_KF * _CH,), jnp.int32),
            [pltpu.VMEM((_CH, _D), jnp.float32) for _ in range(_NB)],
            pltpu.VMEM_SHARED((_NP, _D), jnp.float32),
            [pltpu.SemaphoreType.DMA for _ in range(_NB)],
            [pltpu.SemaphoreType.DMA for _ in range(_NB)],
        ],
    )
    def run(ml_hbm, src_hbm, dst_hbm, zz_hbm, out_hbm,
            src_v, dst_v, rows, agg_sh, gsems, ssems):
        c = lax.axis_index("c")
        s = lax.axis_index("s")
        wid = s * _NC + c
        kc = lax.select(c == _SLOWC, _KS, _KF)
        pltpu.sync_copy(src_hbm.at[wid], src_v)
        pltpu.sync_copy(dst_hbm.at[wid], dst_v)
        pltpu.sync_copy(zz_hbm.at[pl.ds(s * _RPT, _RPT)],
                        agg_sh.at[pl.ds(s * _RPT, _RPT)])
        for b in range(_NB):
            pltpu.async_copy(
                ml_hbm.at[src_v.at[pl.ds(b * _CH, _CH)]], rows[b], gsems[b])
        plsc.subcore_barrier()

        def step(t, carry):
            jj = t * _NB
            for b in range(_NB):
                j = jj + b
                pltpu.make_async_copy(
                    ml_hbm.at[src_v.at[pl.ds(j * _CH, _CH)]],
                    rows[b], gsems[b]).wait()
                pltpu.async_copy(
                    rows[b], agg_sh.at[dst_v.at[pl.ds(j * _CH, _CH)]],
                    ssems[b], add=True)
            for b in range(_NB):
                j = jj + b
                pltpu.make_async_copy(
                    rows[b], agg_sh.at[dst_v.at[pl.ds(j * _CH, _CH)]],
                    ssems[b]).wait()
                nxt = j + _NB

                @pl.when(nxt < kc)
                def _():
                    pltpu.async_copy(
                        ml_hbm.at[src_v.at[pl.ds(nxt * _CH, _CH)]],
                        rows[b], gsems[b])
            return carry

        lax.fori_loop(0, kc // _NB, step, 0)
        plsc.subcore_barrier()
        pltpu.sync_copy(agg_sh.at[pl.ds(s * _RPT, _RPT)],
                        out_hbm.at[c, pl.ds(s * _RPT, _RPT)])

    return run(ml, src_t, dst_t, zeros_nd)


def _sc_degree(dst_t, zeros_nd, ones_t):
    @functools.partial(
        pl.kernel,
        out_type=jax.ShapeDtypeStruct((_NC, _NP, _D), jnp.float32),
        mesh=_sc_mesh(),
        scratch_types=[
            pltpu.VMEM((_KD, _CHD), jnp.int32),
            pltpu.VMEM((_CHD, _D), jnp.float32),
            pltpu.VMEM_SHARED((_NP, _D), jnp.float32),
            [pltpu.SemaphoreType.DMA for _ in range(_NBD)],
        ],
    )
    def run(dst_hbm, zz_hbm, ones_hbm, out_hbm, dst_v, ones_v, deg_sh, ssems):
        c = lax.axis_index("c")
        s = lax.axis_index("s")
        wid = s * _NC + c
        pltpu.sync_copy(dst_hbm.at[wid], dst_v)
        pltpu.sync_copy(ones_hbm, ones_v)
        pltpu.sync_copy(zz_hbm.at[pl.ds(s * _RPT, _RPT)],
                        deg_sh.at[pl.ds(s * _RPT, _RPT)])
        plsc.subcore_barrier()
        for b in range(_NBD):
            pltpu.async_copy(ones_v, deg_sh.at[dst_v.at[b]], ssems[b],
                             add=True)

        def step(t, carry):
            jj = t * _NBD
            for b in range(_NBD):
                j = jj + b
                pltpu.make_async_copy(
                    ones_v, deg_sh.at[dst_v.at[j]], ssems[b]).wait()
                nxt = j + _NBD

                @pl.when(nxt < _KD)
                def _():
                    pltpu.async_copy(
                        ones_v, deg_sh.at[dst_v.at[nxt]], ssems[b], add=True)
            return carry

        lax.fori_loop(0, _KD // _NBD, step, 0)
        plsc.subcore_barrier()
        pltpu.sync_copy(deg_sh.at[pl.ds(s * _RPT, _RPT)],
                        out_hbm.at[c, pl.ds(s * _RPT, _RPT)])

    return run(dst_t, zeros_nd, ones_t)


def _tc_linear(h, Wl, Wr, b2):
    def body(h_ref, wl_ref, wr_ref, b_ref, ml_ref, hr_ref):
        hh = h_ref[...]
        ml_ref[...] = jnp.dot(hh, wl_ref[...],
                              preferred_element_type=jnp.float32)
        hr_ref[...] = jnp.dot(hh, wr_ref[...],
                              preferred_element_type=jnp.float32) + b_ref[...]

    return pl.pallas_call(
        body,
        out_shape=(jax.ShapeDtypeStruct((_N, _D), jnp.float32),
                   jax.ShapeDtypeStruct((_N, _D), jnp.float32)),
    )(h, Wl, Wr, b2)


def _tc_combine(a0, a1, d0, d1, hr):
    def body(a0_ref, a1_ref, d0_ref, d1_ref, hr_ref, o_ref):
        agg = a0_ref[0:_N, :] + a1_ref[0:_N, :]
        deg = d0_ref[0:_N, 0:1] + d1_ref[0:_N, 0:1]
        rdeg = 1.0 / jnp.maximum(deg, 1.0)
        o_ref[...] = jnp.maximum(agg * rdeg + hr_ref[...], 0.0)

    return pl.pallas_call(
        body,
        out_shape=jax.ShapeDtypeStruct((_N, _D), jnp.float32),
    )(a0, a1, d0, d1, hr)


def _tc_combine_linear(a0, a1, d0, d1, hr, Wl, Wr, b2):
    def body(a0_ref, a1_ref, d0_ref, d1_ref, hr_ref, wl_ref, wr_ref, b_ref,
             ml_ref, hr2_ref):
        agg = a0_ref[0:_N, :] + a1_ref[0:_N, :]
        deg = d0_ref[0:_N, 0:1] + d1_ref[0:_N, 0:1]
        rdeg = 1.0 / jnp.maximum(deg, 1.0)
        h = jnp.maximum(agg * rdeg + hr_ref[...], 0.0)
        ml_ref[...] = jnp.dot(h, wl_ref[...],
                              preferred_element_type=jnp.float32)
        hr2_ref[...] = jnp.dot(h, wr_ref[...],
                               preferred_element_type=jnp.float32) + b_ref[...]

    return pl.pallas_call(
        body,
        out_shape=(jax.ShapeDtypeStruct((_N, _D), jnp.float32),
                   jax.ShapeDtypeStruct((_N, _D), jnp.float32)),
    )(a0, a1, d0, d1, hr, Wl, Wr, b2)


def kernel(x, edge_index, Wl0, Wr0, b0, Wl1, Wr1, b1,
           Wl2, Wr2, b2, Wl3, Wr3, b3):
    src = edge_index[0]
    dst = edge_index[1]

    def _layout(idx, fill):
        pad = _EPAD - _E
        flat = jnp.concatenate([idx, jnp.full((pad,), fill, jnp.int32)])
        ls = _KS * _CH
        lf = _KF * _CH
        slow = flat[:_NS * ls].reshape(_NS, ls)
        slow = jnp.concatenate(
            [slow, jnp.full((_NS, lf - ls), fill, jnp.int32)], axis=1)
        fast = flat[_NS * ls:].reshape(_NS, lf)
        parts = (slow, fast) if _SLOWC == 0 else (fast, slow)
        return jnp.stack(parts, axis=1).reshape(_NW, lf)

    src_t = _layout(src, 0)
    dst_t = _layout(dst, _N)
    padd = _EPADD - _E
    dst_td = jnp.concatenate(
        [dst, jnp.full((padd,), _N, jnp.int32)]).reshape(_NW, _KD, _CHD)
    zeros_nd = jnp.zeros((_NP, _D), jnp.float32)
    ones_t = jnp.ones((_CHD, _D), jnp.float32)

    dg = _sc_degree(dst_td, zeros_nd, ones_t)

    ml, hr = _tc_linear(x, Wl0, Wr0, b0.reshape(1, _D))
    nxt = ((Wl1, Wr1, b1), (Wl2, Wr2, b2), (Wl3, Wr3, b3))
    for i in range(4):
        a = _sc_aggregate(ml, src_t, dst_t, zeros_nd)
        if i < 3:
            Wl, Wr, b = nxt[i]
            ml, hr = _tc_combine_linear(a[0], a[1], dg[0], dg[1], hr,
                                        Wl, Wr, b.reshape(1, _D))
        else:
            h = _tc_combine(a[0], a[1], dg[0], dg[1], hr)
    return h


# speedup vs baseline: 1.8768x; 1.6590x over previous
import functools

import jax
import jax.numpy as jnp
from jax import lax
from jax.experimental import pallas as pl
from jax.experimental.pallas import tpu as pltpu
from jax.experimental.pallas import tpu_sc as plsc

_N = 10000
_E = 320000
_D = 128
_NC = 2
_NS = 16
_NW = _NC * _NS
_NB = 5
_CH = 40
_SLOWC = 0
_KS = 250
_KF = 250
_EPAD = _NS * (_KS + _KF) * _CH
_CHD = 128
_KD = 80
_NBD = 4
_EPADD = _NW * _KD * _CHD
_NP = 10112
_RPT = _NP // _NS


def _sc_mesh():
    return plsc.VectorSubcoreMesh(
        core_axis_name="c", subcore_axis_name="s",
        num_cores=_NC, num_subcores=_NS)


def _sc_aggregate(ml, src_t, dst_t, zeros_nd):
    @functools.partial(
        pl.kernel,
        out_type=jax.ShapeDtypeStruct((_NC, _NP, _D), jnp.float32),
        mesh=_sc_mesh(),
        scratch_types=[
            pltpu.VMEM((_KF * _CH,), jnp.int32),
            pltpu.VMEM((_KF * _CH,), jnp.int32),
            [pltpu.VMEM((_CH, _D), jnp.float32) for _ in range(_NB)],
            pltpu.VMEM_SHARED((_NP, _D), jnp.float32),
            [pltpu.SemaphoreType.DMA for _ in range(_NB)],
            [pltpu.SemaphoreType.DMA for _ in range(_NB)],
        ],
    )
    def run(ml_hbm, src_hbm, dst_hbm, zz_hbm, out_hbm,
            src_v, dst_v, rows, agg_sh, gsems, ssems):
        c = lax.axis_index("c")
        s = lax.axis_index("s")
        wid = s * _NC + c
        kc = lax.select(c == _SLOWC, _KS, _KF)
        pltpu.sync_copy(src_hbm.at[wid], src_v)
        pltpu.sync_copy(dst_hbm.at[wid], dst_v)
        pltpu.sync_copy(zz_hbm.at[pl.ds(s * _RPT, _RPT)],
                        agg_sh.at[pl.ds(s * _RPT, _RPT)])
        for b in range(_NB):
            pltpu.async_copy(
                ml_hbm.at[src_v.at[pl.ds(b * _CH, _CH)]], rows[b], gsems[b])
        plsc.subcore_barrier()

        def step(t, carry):
            jj = t * _NB
            for b in range(_NB):
                j = jj + b
                pltpu.make_async_copy(
                    ml_hbm.at[src_v.at[pl.ds(j * _CH, _CH)]],
                    rows[b], gsems[b]).wait()
                pltpu.async_copy(
                    rows[b], agg_sh.at[dst_v.at[pl.ds(j * _CH, _CH)]],
                    ssems[b], add=True)
            for b in range(_NB):
                j = jj + b
                pltpu.make_async_copy(
                    rows[b], agg_sh.at[dst_v.at[pl.ds(j * _CH, _CH)]],
                    ssems[b]).wait()
                nxt = j + _NB

                @pl.when(nxt < kc)
                def _():
                    pltpu.async_copy(
                        ml_hbm.at[src_v.at[pl.ds(nxt * _CH, _CH)]],
                        rows[b], gsems[b])
            return carry

        lax.fori_loop(0, kc // _NB, step, 0)
        plsc.subcore_barrier()
        pltpu.sync_copy(agg_sh.at[pl.ds(s * _RPT, _RPT)],
                        out_hbm.at[c, pl.ds(s * _RPT, _RPT)])

    return run(ml, src_t, dst_t, zeros_nd)


def _sc_degree(dst_t, zeros_nd, ones_t):
    @functools.partial(
        pl.kernel,
        out_type=jax.ShapeDtypeStruct((_NC, _NP, _D), jnp.float32),
        mesh=_sc_mesh(),
        scratch_types=[
            pltpu.VMEM((_KD, _CHD), jnp.int32),
            pltpu.VMEM((_CHD, _D), jnp.float32),
            pltpu.VMEM_SHARED((_NP, _D), jnp.float32),
            [pltpu.SemaphoreType.DMA for _ in range(_NBD)],
        ],
    )
    def run(dst_hbm, zz_hbm, ones_hbm, out_hbm, dst_v, ones_v, deg_sh, ssems):
        c = lax.axis_index("c")
        s = lax.axis_index("s")
        wid = s * _NC + c
        pltpu.sync_copy(dst_hbm.at[wid], dst_v)
        pltpu.sync_copy(ones_hbm, ones_v)
        pltpu.sync_copy(zz_hbm.at[pl.ds(s * _RPT, _RPT)],
                        deg_sh.at[pl.ds(s * _RPT, _RPT)])
        plsc.subcore_barrier()
        for b in range(_NBD):
            pltpu.async_copy(ones_v, deg_sh.at[dst_v.at[b]], ssems[b],
                             add=True)

        def step(t, carry):
            jj = t * _NBD
            for b in range(_NBD):
                j = jj + b
                pltpu.make_async_copy(
                    ones_v, deg_sh.at[dst_v.at[j]], ssems[b]).wait()
                nxt = j + _NBD

                @pl.when(nxt < _KD)
                def _():
                    pltpu.async_copy(
                        ones_v, deg_sh.at[dst_v.at[nxt]], ssems[b], add=True)
            return carry

        lax.fori_loop(0, _KD // _NBD, step, 0)
        plsc.subcore_barrier()
        pltpu.sync_copy(deg_sh.at[pl.ds(s * _RPT, _RPT)],
                        out_hbm.at[c, pl.ds(s * _RPT, _RPT)])

    return run(dst_t, zeros_nd, ones_t)


def _tc_linear(h, Wl, Wr, b2):
    def body(h_ref, wl_ref, wr_ref, b_ref, ml_ref, hr_ref):
        hh = h_ref[...]
        ml_ref[...] = jnp.dot(hh, wl_ref[...],
                              preferred_element_type=jnp.float32)
        hr_ref[...] = jnp.dot(hh, wr_ref[...],
                              preferred_element_type=jnp.float32) + b_ref[...]

    return pl.pallas_call(
        body,
        out_shape=(jax.ShapeDtypeStruct((_N, _D), jnp.float32),
                   jax.ShapeDtypeStruct((_N, _D), jnp.float32)),
    )(h, Wl, Wr, b2)


def _tc_combine(a0, a1, d0, d1, hr):
    def body(a0_ref, a1_ref, d0_ref, d1_ref, hr_ref, o_ref):
        agg = a0_ref[0:_N, :] + a1_ref[0:_N, :]
        deg = d0_ref[0:_N, 0:1] + d1_ref[0:_N, 0:1]
        rdeg = 1.0 / jnp.maximum(deg, 1.0)
        o_ref[...] = jnp.maximum(agg * rdeg + hr_ref[...], 0.0)

    return pl.pallas_call(
        body,
        out_shape=jax.ShapeDtypeStruct((_N, _D), jnp.float32),
    )(a0, a1, d0, d1, hr)


def _tc_combine_linear(a0, a1, d0, d1, hr, Wl, Wr, b2):
    def body(a0_ref, a1_ref, d0_ref, d1_ref, hr_ref, wl_ref, wr_ref, b_ref,
             ml_ref, hr2_ref):
        agg = a0_ref[0:_N, :] + a1_ref[0:_N, :]
        deg = d0_ref[0:_N, 0:1] + d1_ref[0:_N, 0:1]
        rdeg = 1.0 / jnp.maximum(deg, 1.0)
        h = jnp.maximum(agg * rdeg + hr_ref[...], 0.0)
        ml_ref[...] = jnp.dot(h, wl_ref[...],
                              preferred_element_type=jnp.float32)
        hr2_ref[...] = jnp.dot(h, wr_ref[...],
                               preferred_element_type=jnp.float32) + b_ref[...]

    return pl.pallas_call(
        body,
        out_shape=(jax.ShapeDtypeStruct((_N, _D), jnp.float32),
                   jax.ShapeDtypeStruct((_N, _D), jnp.float32)),
    )(a0, a1, d0, d1, hr, Wl, Wr, b2)


def kernel(x, edge_index, Wl0, Wr0, b0, Wl1, Wr1, b1,
           Wl2, Wr2, b2, Wl3, Wr3, b3):
    src = edge_index[0]
    dst = edge_index[1]

    def _layout(idx, fill):
        pad = _EPAD - _E
        flat = jnp.concatenate([idx, jnp.full((pad,), fill, jnp.int32)])
        ls = _KS * _CH
        lf = _KF * _CH
        slow = flat[:_NS * ls].reshape(_NS, ls)
        slow = jnp.concatenate(
            [slow, jnp.full((_NS, lf - ls), fill, jnp.int32)], axis=1)
        fast = flat[_NS * ls:].reshape(_NS, lf)
        parts = (slow, fast) if _SLOWC == 0 else (fast, slow)
        return jnp.stack(parts, axis=1).reshape(_NW, lf)

    src_t = _layout(src, 0)
    dst_t = _layout(dst, _N)
    padd = _EPADD - _E
    dst_td = jnp.concatenate(
        [dst, jnp.full((padd,), _N, jnp.int32)]).reshape(_NW, _KD, _CHD)
    zeros_nd = jnp.zeros((_NP, _D), jnp.float32)
    ones_t = jnp.ones((_CHD, _D), jnp.float32)

    dg = _sc_degree(dst_td, zeros_nd, ones_t)

    ml, hr = _tc_linear(x, Wl0, Wr0, b0.reshape(1, _D))
    nxt = ((Wl1, Wr1, b1), (Wl2, Wr2, b2), (Wl3, Wr3, b3))
    for i in range(4):
        a = _sc_aggregate(ml, src_t, dst_t, zeros_nd)
        if i < 3:
            Wl, Wr, b = nxt[i]
            ml, hr = _tc_combine_linear(a[0], a[1], dg[0], dg[1], hr,
                                        Wl, Wr, b.reshape(1, _D))
        else:
            h = _tc_combine(a[0], a[1], dg[0], dg[1], hr)
    return h
